# baseline (speedup 1.0000x reference)
import functools

import jax
import jax.numpy as jnp
from jax import lax
from jax.experimental import pallas as pl
from jax.experimental.pallas import tpu as pltpu
from jax.experimental.pallas import tpu_sc as plsc

_LW = 8
_SC_CHUNK = 256
_NW = 32


def _embed_gather(x_flat, embed):
    ntok = x_flat.shape[0]
    d = embed.shape[1]
    bpw = ntok // _NW
    mesh = plsc.VectorSubcoreMesh(core_axis_name="c", subcore_axis_name="s")

    @functools.partial(
        pl.kernel,
        mesh=mesh,
        out_type=jax.ShapeDtypeStruct((ntok, d), jnp.float32),
        scratch_types=[
            pltpu.VMEM((bpw,), jnp.int32),
            pltpu.VMEM((bpw, d), jnp.float32),
            pltpu.SemaphoreType.DMA,
        ],
    )
    def gather_kernel(table_hbm, idx_hbm, out_hbm, idx_v, rows_v, sem):
        wid = lax.axis_index("s") * 2 + lax.axis_index("c")
        base = wid * bpw
        pltpu.sync_copy(idx_hbm.at[pl.ds(base, bpw)], idx_v)
        pltpu.async_copy(table_hbm.at[idx_v], rows_v, sem).wait()
        pltpu.sync_copy(rows_v, out_hbm.at[pl.ds(base, bpw)])

    return gather_kernel(embed, x_flat)


def _tc_body(h_ref, td_ref, wk_ref, wv_ref, wr_ref, mkt_ref, mv_ref,
             wq_ref, wo_ref, ow_ref, ob_ref, out_ref,
             dm1_ref, dm2_ref, dm4_ref, dnx_ref, a_ref, b_ref, h2_scr):
    bsz = h_ref.shape[0]
    dmodel = h_ref.shape[2]
    vocab = out_ref.shape[1]
    cap = 50
    rows = bsz * _SC_CHUNK
    nw = _SC_CHUNK // _LW
    f32 = jnp.float32

    ne = jnp.exp(td_ref[...])

    @pl.when(pl.program_id(0) == 0)
    def _init():
        a_ref[...] = jnp.zeros_like(a_ref)
        b_ref[...] = jnp.zeros_like(b_ref)
        h2_scr[...] = jnp.zeros_like(h2_scr)
        rowmod = (lax.broadcasted_iota(jnp.int32, (rows, dmodel), 0) & (_LW - 1)
                  ).astype(f32)
        dm1_ref[...] = jnp.where(rowmod >= 1, jnp.exp(-1.0 * ne), 0.0)
        dm2_ref[...] = jnp.where(rowmod >= 2, jnp.exp(-2.0 * ne), 0.0)
        dm4_ref[...] = jnp.where(rowmod >= 4, jnp.exp(-4.0 * ne), 0.0)
        dnx_ref[...] = jnp.exp(-(rowmod + 1.0) * ne)

    hm = h_ref[...].reshape(rows, dmodel)
    k2 = jnp.dot(hm, wk_ref[...], preferred_element_type=f32)
    v2 = jnp.dot(hm, wv_ref[...], preferred_element_type=f32)
    r2 = 1.0 / (1.0 + jnp.exp(-jnp.dot(hm, wr_ref[...],
                                       preferred_element_type=f32)))

    ek = jnp.exp(jnp.clip(k2, -30.0, 30.0))
    ua = ek * v2
    ub = ek
    for dm_ref, sh in ((dm1_ref, 1), (dm2_ref, 2), (dm4_ref, 4)):
        dm = dm_ref[...]
        z = jnp.zeros((sh, dmodel), f32)
        ua = ua + dm * jnp.concatenate([z, ua[:rows - sh, :]], axis=0)
        ub = ub + dm * jnp.concatenate([z, ub[:rows - sh, :]], axis=0)

    ta = ua.reshape(rows // _LW, _LW, dmodel)[:, _LW - 1, :]
    tb = ub.reshape(rows // _LW, _LW, dmodel)[:, _LW - 1, :]
    ta = ta.reshape(bsz, nw, dmodel)
    tb = tb.reshape(bsz, nw, dmodel)
    sh = 1
    while sh < nw:
        dsh = jnp.exp(f32(-_LW * sh) * ne)
        z = jnp.zeros((bsz, sh, dmodel), f32)
        ta = ta + dsh * jnp.concatenate([z, ta[:, :nw - sh, :]], axis=1)
        tb = tb + dsh * jnp.concatenate([z, tb[:, :nw - sh, :]], axis=1)
        sh *= 2
    jpos = (lax.broadcasted_iota(jnp.int32, (nw, dmodel), 0) + 1).astype(f32)
    p8 = jnp.exp((-float(_LW) * jpos) * ne)[None]
    a_in = a_ref[...]
    b_in = b_ref[...]
    afull = p8 * a_in[:, None, :] + ta
    bfull = p8 * b_in[:, None, :] + tb
    pid = pl.program_id(0)
    live = pid < pl.num_programs(0) - 1
    a_ref[...] = jnp.where(live, afull[:, nw - 1, :], a_in)
    b_ref[...] = jnp.where(live, bfull[:, nw - 1, :], b_in)
    aprev = jnp.concatenate([a_in[:, None, :], afull[:, :nw - 1, :]], axis=1)
    bprev = jnp.concatenate([b_in[:, None, :], bfull[:, :nw - 1, :]], axis=1)
    aex = jnp.broadcast_to(aprev.reshape(rows // _LW, 1, dmodel),
                           (rows // _LW, _LW, dmodel)).reshape(rows, dmodel)
    bex = jnp.broadcast_to(bprev.reshape(rows // _LW, 1, dmodel),
                           (rows // _LW, _LW, dmodel)).reshape(rows, dmodel)
    dnx = dnx_ref[...]
    wkv = (dnx * aex + ua) / (dnx * bex + ub + 1e-8)
    h2_scr[pid % 2] = hm + r2 * wkv

    h2 = h2_scr[(pid + 1) % 2]

    q = jnp.dot(h2, wq_ref[...], preferred_element_type=f32)
    scores = jnp.dot(q, mkt_ref[...], preferred_element_type=f32)
    scores = scores / jnp.sqrt(f32(dmodel))
    colid = lax.broadcasted_iota(jnp.int32, (rows, mkt_ref.shape[1]), 1
                                 ).astype(f32)
    neg = f32(-1e30)
    scores = jnp.where(colid < cap, scores, neg)
    m1 = jnp.max(scores, axis=1, keepdims=True)
    i1 = jnp.min(jnp.where(scores == m1, colid, f32(1e9)), axis=1, keepdims=True)
    mask1 = colid == i1
    s2 = jnp.where(mask1, neg, scores)
    m2 = jnp.max(s2, axis=1, keepdims=True)
    i2 = jnp.min(jnp.where(s2 == m2, colid, f32(1e9)), axis=1, keepdims=True)
    mask2 = colid == i2
    e2 = jnp.exp(m2 - m1)
    inv = 1.0 / (1.0 + e2)
    wfull = jnp.where(mask1, inv, 0.0) + jnp.where(mask2, e2 * inv, 0.0)
    retrieved = jnp.dot(wfull, mv_ref[...], preferred_element_type=f32)
    h3 = h2 + jnp.dot(retrieved, wo_ref[...], preferred_element_type=f32)

    outt = jnp.dot(ow_ref[...], h3.T, preferred_element_type=f32) + ob_ref[...]
    for b in range(bsz):
        out_ref[b] = outt[:, b * _SC_CHUNK:(b + 1) * _SC_CHUNK]


def kernel(x, embed, time_decay, Wk, Wv, Wr, mem_keys, mem_vals, Wq, Wo,
           out_W, out_b):
    bsz, seq = x.shape
    vocab, dmodel = embed.shape
    cap = mem_keys.shape[0]
    capp = 128

    h = _embed_gather(x.reshape(-1).astype(jnp.int32), embed)
    h = h.reshape(bsz, seq, dmodel)

    mkt = jnp.zeros((dmodel, capp), jnp.float32).at[:, :cap].set(mem_keys.T)
    mv = jnp.zeros((capp, dmodel), jnp.float32).at[:cap].set(mem_vals)
    td2 = time_decay.reshape(1, dmodel)
    owt = out_W.T
    obc = out_b.reshape(vocab, 1)

    out = _tc_call(h, td2, Wk, Wv, Wr, mkt, mv, Wq, Wo, owt, obc)
    return jnp.swapaxes(out, 1, 2)


def _tc_call(h, td2, Wk, Wv, Wr, mkt, mv, Wq, Wo, owt, obc, interpret=False):
    bsz, seq, dmodel = h.shape
    vocab = owt.shape[0]
    capp = mkt.shape[1]
    nblk = seq // _SC_CHUNK
    grid = (nblk + 1,)
    return pl.pallas_call(
        _tc_body,
        grid=grid,
        in_specs=[
            pl.BlockSpec((bsz, _SC_CHUNK, dmodel),
                         lambda i: (0, jnp.minimum(i, nblk - 1), 0)),
            pl.BlockSpec((1, dmodel), lambda i: (0, 0)),
            pl.BlockSpec((dmodel, dmodel), lambda i: (0, 0)),
            pl.BlockSpec((dmodel, dmodel), lambda i: (0, 0)),
            pl.BlockSpec((dmodel, dmodel), lambda i: (0, 0)),
            pl.BlockSpec((dmodel, capp), lambda i: (0, 0)),
            pl.BlockSpec((capp, dmodel), lambda i: (0, 0)),
            pl.BlockSpec((dmodel, dmodel), lambda i: (0, 0)),
            pl.BlockSpec((dmodel, dmodel), lambda i: (0, 0)),
            pl.BlockSpec((vocab, dmodel), lambda i: (0, 0)),
            pl.BlockSpec((vocab, 1), lambda i: (0, 0)),
        ],
        out_specs=pl.BlockSpec((bsz, vocab, _SC_CHUNK),
                               lambda i: (0, 0, jnp.maximum(i - 1, 0))),
        out_shape=jax.ShapeDtypeStruct((bsz, vocab, seq), jnp.float32),
        scratch_shapes=[
            pltpu.VMEM((bsz * _SC_CHUNK, dmodel), jnp.float32),
            pltpu.VMEM((bsz * _SC_CHUNK, dmodel), jnp.float32),
            pltpu.VMEM((bsz * _SC_CHUNK, dmodel), jnp.float32),
            pltpu.VMEM((bsz * _SC_CHUNK, dmodel), jnp.float32),
            pltpu.VMEM((bsz, dmodel), jnp.float32),
            pltpu.VMEM((bsz, dmodel), jnp.float32),
            pltpu.VMEM((2, bsz * _SC_CHUNK, dmodel), jnp.float32),
        ],
        interpret=interpret,
    )(h, td2, Wk, Wv, Wr, mkt, mv, Wq, Wo, owt, obc)


# speedup vs baseline: 1.1211x; 1.1211x over previous
import functools

import jax
import jax.numpy as jnp
from jax import lax
from jax.experimental import pallas as pl
from jax.experimental.pallas import tpu as pltpu
from jax.experimental.pallas import tpu_sc as plsc

_LW = 8
_SC_CHUNK = 256
_NW = 32


def _embed_gather(x_flat, embed):
    ntok = x_flat.shape[0]
    d = embed.shape[1]
    bpw = ntok // _NW
    mesh = plsc.VectorSubcoreMesh(core_axis_name="c", subcore_axis_name="s")

    @functools.partial(
        pl.kernel,
        mesh=mesh,
        out_type=jax.ShapeDtypeStruct((ntok, d), jnp.float32),
        scratch_types=[
            pltpu.VMEM((bpw,), jnp.int32),
            pltpu.VMEM((bpw, d), jnp.float32),
            pltpu.SemaphoreType.DMA,
        ],
    )
    def gather_kernel(table_hbm, idx_hbm, out_hbm, idx_v, rows_v, sem):
        wid = lax.axis_index("s") * 2 + lax.axis_index("c")
        base = wid * bpw
        pltpu.sync_copy(idx_hbm.at[pl.ds(base, bpw)], idx_v)
        pltpu.async_copy(table_hbm.at[idx_v], rows_v, sem).wait()
        pltpu.sync_copy(rows_v, out_hbm.at[pl.ds(base, bpw)])

    return gather_kernel(embed, x_flat)


def _tc_body(h_ref, td_ref, wk_ref, wv_ref, wr_ref, mkt_ref, mv_ref,
             wq_ref, wo_ref, ow_ref, ob_ref, out_ref,
             dnx_ref, a_ref, b_ref):
    bsz = h_ref.shape[0]
    dmodel = h_ref.shape[2]
    vocab = out_ref.shape[1]
    cap = 50
    rows = bsz * _SC_CHUNK
    nwin = rows // _LW
    nw = _SC_CHUNK // _LW
    f32 = jnp.float32

    ne = jnp.exp(td_ref[...])

    @pl.when(pl.program_id(0) == 0)
    def _init():
        a_ref[...] = jnp.zeros_like(a_ref)
        b_ref[...] = jnp.zeros_like(b_ref)
        rowmod = (lax.broadcasted_iota(jnp.int32, (rows, dmodel), 0) & (_LW - 1)
                  ).astype(f32)
        dnx_ref[...] = jnp.exp(-(rowmod + 1.0) * ne)

    hm = h_ref[...].reshape(rows, dmodel)
    k2 = jnp.dot(hm, wk_ref[...], preferred_element_type=f32)
    v2 = jnp.dot(hm, wv_ref[...], preferred_element_type=f32)
    r2 = 1.0 / (1.0 + jnp.exp(-jnp.dot(hm, wr_ref[...],
                                       preferred_element_type=f32)))

    ek = jnp.exp(jnp.clip(k2, -30.0, 30.0))
    ua = (ek * v2).reshape(nwin, _LW, dmodel)
    ub = ek.reshape(nwin, _LW, dmodel)
    sh = 1
    while sh < _LW:
        dsh = jnp.exp(f32(-sh) * ne)[None]
        z = jnp.zeros((nwin, sh, dmodel), f32)
        ua = ua + dsh * jnp.concatenate([z, ua[:, :_LW - sh, :]], axis=1)
        ub = ub + dsh * jnp.concatenate([z, ub[:, :_LW - sh, :]], axis=1)
        sh *= 2

    ngrp = nwin // _LW
    w2a = ua[:, _LW - 1, :].reshape(ngrp, _LW, dmodel)
    w2b = ub[:, _LW - 1, :].reshape(ngrp, _LW, dmodel)
    sh = 1
    while sh < _LW:
        dsh = jnp.exp(f32(-_LW * sh) * ne)[None]
        z = jnp.zeros((ngrp, sh, dmodel), f32)
        w2a = w2a + dsh * jnp.concatenate([z, w2a[:, :_LW - sh, :]], axis=1)
        w2b = w2b + dsh * jnp.concatenate([z, w2b[:, :_LW - sh, :]], axis=1)
        sh *= 2

    npb = ngrp // bsz
    gl = _LW * _LW
    ga = w2a[:, _LW - 1, :].reshape(bsz, npb, dmodel)
    gb = w2b[:, _LW - 1, :].reshape(bsz, npb, dmodel)
    sh = 1
    while sh < npb:
        dsh = jnp.exp(f32(-gl * sh) * ne)[None]
        z = jnp.zeros((bsz, sh, dmodel), f32)
        ga = ga + dsh * jnp.concatenate([z, ga[:, :npb - sh, :]], axis=1)
        gb = gb + dsh * jnp.concatenate([z, gb[:, :npb - sh, :]], axis=1)
        sh *= 2

    a_in = a_ref[...]
    b_in = b_ref[...]
    p_all = jnp.exp(f32(-gl * npb) * ne)
    a_ref[...] = p_all * a_in + ga[:, npb - 1, :]
    b_ref[...] = p_all * b_in + gb[:, npb - 1, :]

    kpos = lax.broadcasted_iota(jnp.int32, (npb, dmodel), 0).astype(f32)
    p64 = jnp.exp((-f32(gl) * kpos) * ne)[None]
    zb = jnp.zeros((bsz, 1, dmodel), f32)
    sga = p64 * a_in[:, None, :] + jnp.concatenate(
        [zb, ga[:, :npb - 1, :]], axis=1)
    sgb = p64 * b_in[:, None, :] + jnp.concatenate(
        [zb, gb[:, :npb - 1, :]], axis=1)

    jpos = lax.broadcasted_iota(jnp.int32, (_LW, dmodel), 0).astype(f32)
    pw8 = jnp.exp((-f32(_LW) * jpos) * ne)[None]
    zg = jnp.zeros((ngrp, 1, dmodel), f32)
    sga_e = jnp.broadcast_to(sga.reshape(ngrp, 1, dmodel), (ngrp, _LW, dmodel))
    sgb_e = jnp.broadcast_to(sgb.reshape(ngrp, 1, dmodel), (ngrp, _LW, dmodel))
    swa = pw8 * sga_e + jnp.concatenate([zg, w2a[:, :_LW - 1, :]], axis=1)
    swb = pw8 * sgb_e + jnp.concatenate([zg, w2b[:, :_LW - 1, :]], axis=1)

    aex = jnp.broadcast_to(swa.reshape(nwin, 1, dmodel), (nwin, _LW, dmodel))
    bex = jnp.broadcast_to(swb.reshape(nwin, 1, dmodel), (nwin, _LW, dmodel))
    dnx = dnx_ref[...].reshape(nwin, _LW, dmodel)
    wkv = ((dnx * aex + ua) / (dnx * bex + ub + 1e-8)).reshape(rows, dmodel)
    h2 = hm + r2 * wkv

    q = jnp.dot(h2, wq_ref[...], preferred_element_type=f32)
    scores = jnp.dot(q, mkt_ref[...], preferred_element_type=f32)
    scores = scores / jnp.sqrt(f32(dmodel))
    colid = lax.broadcasted_iota(jnp.int32, (rows, mkt_ref.shape[1]), 1
                                 ).astype(f32)
    neg = f32(-1e30)
    scores = jnp.where(colid < cap, scores, neg)
    m1 = jnp.max(scores, axis=1, keepdims=True)
    i1 = jnp.min(jnp.where(scores == m1, colid, f32(1e9)), axis=1, keepdims=True)
    mask1 = colid == i1
    s2 = jnp.where(mask1, neg, scores)
    m2 = jnp.max(s2, axis=1, keepdims=True)
    i2 = jnp.min(jnp.where(s2 == m2, colid, f32(1e9)), axis=1, keepdims=True)
    mask2 = colid == i2
    e2 = jnp.exp(m2 - m1)
    inv = 1.0 / (1.0 + e2)
    wfull = jnp.where(mask1, inv, 0.0) + jnp.where(mask2, e2 * inv, 0.0)
    retrieved = jnp.dot(wfull, mv_ref[...], preferred_element_type=f32)
    h3 = h2 + jnp.dot(retrieved, wo_ref[...], preferred_element_type=f32)

    outt = jnp.dot(ow_ref[...], h3.T, preferred_element_type=f32) + ob_ref[...]
    for b in range(bsz):
        out_ref[b] = outt[:, b * _SC_CHUNK:(b + 1) * _SC_CHUNK]


def kernel(x, embed, time_decay, Wk, Wv, Wr, mem_keys, mem_vals, Wq, Wo,
           out_W, out_b):
    bsz, seq = x.shape
    vocab, dmodel = embed.shape
    cap = mem_keys.shape[0]
    capp = 128

    h = _embed_gather(x.reshape(-1).astype(jnp.int32), embed)
    h = h.reshape(bsz, seq, dmodel)

    mkt = jnp.zeros((dmodel, capp), jnp.float32).at[:, :cap].set(mem_keys.T)
    mv = jnp.zeros((capp, dmodel), jnp.float32).at[:cap].set(mem_vals)
    td2 = time_decay.reshape(1, dmodel)
    owt = out_W.T
    obc = out_b.reshape(vocab, 1)

    out = _tc_call(h, td2, Wk, Wv, Wr, mkt, mv, Wq, Wo, owt, obc)
    return jnp.swapaxes(out, 1, 2)


def _tc_call(h, td2, Wk, Wv, Wr, mkt, mv, Wq, Wo, owt, obc, interpret=False):
    bsz, seq, dmodel = h.shape
    vocab = owt.shape[0]
    capp = mkt.shape[1]
    nblk = seq // _SC_CHUNK
    grid = (nblk,)
    return pl.pallas_call(
        _tc_body,
        grid=grid,
        in_specs=[
            pl.BlockSpec((bsz, _SC_CHUNK, dmodel), lambda i: (0, i, 0)),
            pl.BlockSpec((1, dmodel), lambda i: (0, 0)),
            pl.BlockSpec((dmodel, dmodel), lambda i: (0, 0)),
            pl.BlockSpec((dmodel, dmodel), lambda i: (0, 0)),
            pl.BlockSpec((dmodel, dmodel), lambda i: (0, 0)),
            pl.BlockSpec((dmodel, capp), lambda i: (0, 0)),
            pl.BlockSpec((capp, dmodel), lambda i: (0, 0)),
            pl.BlockSpec((dmodel, dmodel), lambda i: (0, 0)),
            pl.BlockSpec((dmodel, dmodel), lambda i: (0, 0)),
            pl.BlockSpec((vocab, dmodel), lambda i: (0, 0)),
            pl.BlockSpec((vocab, 1), lambda i: (0, 0)),
        ],
        out_specs=pl.BlockSpec((bsz, vocab, _SC_CHUNK), lambda i: (0, 0, i)),
        out_shape=jax.ShapeDtypeStruct((bsz, vocab, seq), jnp.float32),
        scratch_shapes=[
            pltpu.VMEM((bsz * _SC_CHUNK, dmodel), jnp.float32),
            pltpu.VMEM((bsz, dmodel), jnp.float32),
            pltpu.VMEM((bsz, dmodel), jnp.float32),
        ],
        interpret=interpret,
    )(h, td2, Wk, Wv, Wr, mkt, mv, Wq, Wo, owt, obc)


# speedup vs baseline: 1.1298x; 1.0078x over previous
import functools

import jax
import jax.numpy as jnp
from jax import lax
from jax.experimental import pallas as pl
from jax.experimental.pallas import tpu as pltpu
from jax.experimental.pallas import tpu_sc as plsc

_LW = 8
_SC_CHUNK = 512
_NW = 32


def _embed_gather(x_flat, embed):
    ntok = x_flat.shape[0]
    d = embed.shape[1]
    bpw = ntok // _NW
    mesh = plsc.VectorSubcoreMesh(core_axis_name="c", subcore_axis_name="s")

    @functools.partial(
        pl.kernel,
        mesh=mesh,
        out_type=jax.ShapeDtypeStruct((ntok, d), jnp.float32),
        scratch_types=[
            pltpu.VMEM((bpw,), jnp.int32),
            pltpu.VMEM((bpw, d), jnp.float32),
            pltpu.SemaphoreType.DMA,
        ],
    )
    def gather_kernel(table_hbm, idx_hbm, out_hbm, idx_v, rows_v, sem):
        wid = lax.axis_index("s") * 2 + lax.axis_index("c")
        base = wid * bpw
        pltpu.sync_copy(idx_hbm.at[pl.ds(base, bpw)], idx_v)
        pltpu.async_copy(table_hbm.at[idx_v], rows_v, sem).wait()
        pltpu.sync_copy(rows_v, out_hbm.at[pl.ds(base, bpw)])

    return gather_kernel(embed, x_flat)


def _tc_body(h_ref, td_ref, wk_ref, wv_ref, wr_ref, mkt_ref, mv_ref,
             wq_ref, wo_ref, ow_ref, ob_ref, out_ref,
             dnx_ref, a_ref, b_ref):
    bsz = h_ref.shape[0]
    dmodel = h_ref.shape[2]
    vocab = out_ref.shape[1]
    cap = 50
    rows = bsz * _SC_CHUNK
    nwin = rows // _LW
    nw = _SC_CHUNK // _LW
    f32 = jnp.float32

    ne = jnp.exp(td_ref[...])

    @pl.when(pl.program_id(0) == 0)
    def _init():
        a_ref[...] = jnp.zeros_like(a_ref)
        b_ref[...] = jnp.zeros_like(b_ref)
        rowmod = (lax.broadcasted_iota(jnp.int32, (rows, dmodel), 0) & (_LW - 1)
                  ).astype(f32)
        dnx_ref[...] = jnp.exp(-(rowmod + 1.0) * ne)

    hm = h_ref[...].reshape(rows, dmodel)
    k2 = jnp.dot(hm, wk_ref[...], preferred_element_type=f32)
    v2 = jnp.dot(hm, wv_ref[...], preferred_element_type=f32)
    r2 = 1.0 / (1.0 + jnp.exp(-jnp.dot(hm, wr_ref[...],
                                       preferred_element_type=f32)))

    ek = jnp.exp(jnp.clip(k2, -30.0, 30.0))
    ua = (ek * v2).reshape(nwin, _LW, dmodel)
    ub = ek.reshape(nwin, _LW, dmodel)
    sh = 1
    while sh < _LW:
        dsh = jnp.exp(f32(-sh) * ne)[None]
        z = jnp.zeros((nwin, sh, dmodel), f32)
        ua = ua + dsh * jnp.concatenate([z, ua[:, :_LW - sh, :]], axis=1)
        ub = ub + dsh * jnp.concatenate([z, ub[:, :_LW - sh, :]], axis=1)
        sh *= 2

    ngrp = nwin // _LW
    w2a = ua[:, _LW - 1, :].reshape(ngrp, _LW, dmodel)
    w2b = ub[:, _LW - 1, :].reshape(ngrp, _LW, dmodel)
    sh = 1
    while sh < _LW:
        dsh = jnp.exp(f32(-_LW * sh) * ne)[None]
        z = jnp.zeros((ngrp, sh, dmodel), f32)
        w2a = w2a + dsh * jnp.concatenate([z, w2a[:, :_LW - sh, :]], axis=1)
        w2b = w2b + dsh * jnp.concatenate([z, w2b[:, :_LW - sh, :]], axis=1)
        sh *= 2

    npb = ngrp // bsz
    gl = _LW * _LW
    ga = w2a[:, _LW - 1, :].reshape(bsz, npb, dmodel)
    gb = w2b[:, _LW - 1, :].reshape(bsz, npb, dmodel)
    sh = 1
    while sh < npb:
        dsh = jnp.exp(f32(-gl * sh) * ne)[None]
        z = jnp.zeros((bsz, sh, dmodel), f32)
        ga = ga + dsh * jnp.concatenate([z, ga[:, :npb - sh, :]], axis=1)
        gb = gb + dsh * jnp.concatenate([z, gb[:, :npb - sh, :]], axis=1)
        sh *= 2

    a_in = a_ref[...]
    b_in = b_ref[...]
    p_all = jnp.exp(f32(-gl * npb) * ne)
    a_ref[...] = p_all * a_in + ga[:, npb - 1, :]
    b_ref[...] = p_all * b_in + gb[:, npb - 1, :]

    kpos = lax.broadcasted_iota(jnp.int32, (npb, dmodel), 0).astype(f32)
    p64 = jnp.exp((-f32(gl) * kpos) * ne)[None]
    zb = jnp.zeros((bsz, 1, dmodel), f32)
    sga = p64 * a_in[:, None, :] + jnp.concatenate(
        [zb, ga[:, :npb - 1, :]], axis=1)
    sgb = p64 * b_in[:, None, :] + jnp.concatenate(
        [zb, gb[:, :npb - 1, :]], axis=1)

    jpos = lax.broadcasted_iota(jnp.int32, (_LW, dmodel), 0).astype(f32)
    pw8 = jnp.exp((-f32(_LW) * jpos) * ne)[None]
    zg = jnp.zeros((ngrp, 1, dmodel), f32)
    sga_e = jnp.broadcast_to(sga.reshape(ngrp, 1, dmodel), (ngrp, _LW, dmodel))
    sgb_e = jnp.broadcast_to(sgb.reshape(ngrp, 1, dmodel), (ngrp, _LW, dmodel))
    swa = pw8 * sga_e + jnp.concatenate([zg, w2a[:, :_LW - 1, :]], axis=1)
    swb = pw8 * sgb_e + jnp.concatenate([zg, w2b[:, :_LW - 1, :]], axis=1)

    aex = jnp.broadcast_to(swa.reshape(nwin, 1, dmodel), (nwin, _LW, dmodel))
    bex = jnp.broadcast_to(swb.reshape(nwin, 1, dmodel), (nwin, _LW, dmodel))
    dnx = dnx_ref[...].reshape(nwin, _LW, dmodel)
    wkv = ((dnx * aex + ua) / (dnx * bex + ub + 1e-8)).reshape(rows, dmodel)
    h2 = hm + r2 * wkv

    q = jnp.dot(h2, wq_ref[...], preferred_element_type=f32)
    scores = jnp.dot(q, mkt_ref[...], preferred_element_type=f32)
    scores = scores / jnp.sqrt(f32(dmodel))
    colid = lax.broadcasted_iota(jnp.int32, (rows, mkt_ref.shape[1]), 1
                                 ).astype(f32)
    neg = f32(-1e30)
    scores = jnp.where(colid < cap, scores, neg)
    m1 = jnp.max(scores, axis=1, keepdims=True)
    i1 = jnp.min(jnp.where(scores == m1, colid, f32(1e9)), axis=1, keepdims=True)
    mask1 = colid == i1
    s2 = jnp.where(mask1, neg, scores)
    m2 = jnp.max(s2, axis=1, keepdims=True)
    i2 = jnp.min(jnp.where(s2 == m2, colid, f32(1e9)), axis=1, keepdims=True)
    mask2 = colid == i2
    e2 = jnp.exp(m2 - m1)
    inv = 1.0 / (1.0 + e2)
    wfull = jnp.where(mask1, inv, 0.0) + jnp.where(mask2, e2 * inv, 0.0)
    retrieved = jnp.dot(wfull, mv_ref[...], preferred_element_type=f32)
    h3 = h2 + jnp.dot(retrieved, wo_ref[...], preferred_element_type=f32)

    outt = jnp.dot(ow_ref[...], h3.T, preferred_element_type=f32) + ob_ref[...]
    for b in range(bsz):
        out_ref[b] = outt[:, b * _SC_CHUNK:(b + 1) * _SC_CHUNK]


def kernel(x, embed, time_decay, Wk, Wv, Wr, mem_keys, mem_vals, Wq, Wo,
           out_W, out_b):
    bsz, seq = x.shape
    vocab, dmodel = embed.shape
    cap = mem_keys.shape[0]
    capp = 128

    h = _embed_gather(x.reshape(-1).astype(jnp.int32), embed)
    h = h.reshape(bsz, seq, dmodel)

    mkt = jnp.zeros((dmodel, capp), jnp.float32).at[:, :cap].set(mem_keys.T)
    mv = jnp.zeros((capp, dmodel), jnp.float32).at[:cap].set(mem_vals)
    td2 = time_decay.reshape(1, dmodel)
    owt = out_W.T
    obc = out_b.reshape(vocab, 1)

    out = _tc_call(h, td2, Wk, Wv, Wr, mkt, mv, Wq, Wo, owt, obc)
    return jnp.swapaxes(out, 1, 2)


def _tc_call(h, td2, Wk, Wv, Wr, mkt, mv, Wq, Wo, owt, obc, interpret=False):
    bsz, seq, dmodel = h.shape
    vocab = owt.shape[0]
    capp = mkt.shape[1]
    nblk = seq // _SC_CHUNK
    grid = (nblk,)
    return pl.pallas_call(
        _tc_body,
        grid=grid,
        in_specs=[
            pl.BlockSpec((bsz, _SC_CHUNK, dmodel), lambda i: (0, i, 0)),
            pl.BlockSpec((1, dmodel), lambda i: (0, 0)),
            pl.BlockSpec((dmodel, dmodel), lambda i: (0, 0)),
            pl.BlockSpec((dmodel, dmodel), lambda i: (0, 0)),
            pl.BlockSpec((dmodel, dmodel), lambda i: (0, 0)),
            pl.BlockSpec((dmodel, capp), lambda i: (0, 0)),
            pl.BlockSpec((capp, dmodel), lambda i: (0, 0)),
            pl.BlockSpec((dmodel, dmodel), lambda i: (0, 0)),
            pl.BlockSpec((dmodel, dmodel), lambda i: (0, 0)),
            pl.BlockSpec((vocab, dmodel), lambda i: (0, 0)),
            pl.BlockSpec((vocab, 1), lambda i: (0, 0)),
        ],
        out_specs=pl.BlockSpec((bsz, vocab, _SC_CHUNK), lambda i: (0, 0, i)),
        out_shape=jax.ShapeDtypeStruct((bsz, vocab, seq), jnp.float32),
        scratch_shapes=[
            pltpu.VMEM((bsz * _SC_CHUNK, dmodel), jnp.float32),
            pltpu.VMEM((bsz, dmodel), jnp.float32),
            pltpu.VMEM((bsz, dmodel), jnp.float32),
        ],
        interpret=interpret,
    )(h, td2, Wk, Wv, Wr, mkt, mv, Wq, Wo, owt, obc)
